# Initial kernel scaffold; baseline (speedup 1.0000x reference)
import jax
import jax.numpy as jnp
from jax.experimental import pallas as pl


def kernel(x, edge_index, edge_attr, edge_type, label_indices, correct_label_mask, W_rel0, W_self0, W_edge0, b0, W_rel1, W_self1, W_edge1, b1, Wq, Wk, Wv, mlp_w1, mlp_b1, mlp_w2, mlp_b2, mlp_w3, mlp_b3):
    raise NotImplementedError("write your pallas kernel here")


# speedup vs baseline: 14.4183x; 14.4183x over previous
import functools

import jax
import jax.numpy as jnp
from jax import lax
from jax.experimental import pallas as pl
from jax.experimental.pallas import tpu as pltpu
from jax.experimental.pallas import tpu_sc as plsc

N = 10000
E = 320000
D = 128
DE = 16
R = 4
L = 512

NC = 2
NS = 16
NW = NC * NS
EPW = E // NW
C = 80
NCH = EPW // C
NP = 10240
NPT = NP // NS

_f32 = jnp.float32
_i32 = jnp.int32


def _zero_vmem_f32(ref, rows, cols):
    z = jnp.zeros((16,), _f32)

    def body(i, _):
        for j in range(cols // 16):
            ref[i, pl.ds(j * 16, 16)] = z
        return 0

    lax.fori_loop(0, rows, body, 0)


def _zero_stripe(zbuf, acc, base):
    for b in range(NPT // C):
        pltpu.sync_copy(zbuf, acc.at[pl.ds(base + b * C, C)])


def _sc_gidx(src2d, typ2d):
    mesh = plsc.VectorSubcoreMesh(core_axis_name="c", subcore_axis_name="s")

    @functools.partial(
        pl.kernel,
        out_type=jax.ShapeDtypeStruct((NW, NCH, C), _i32),
        mesh=mesh,
        scratch_types=[
            pltpu.VMEM((NCH, C), _i32),
            pltpu.VMEM((NCH, C), _i32),
        ],
    )
    def k(src_h, typ_h, gidx_h, gidx_v, typ_v):
        cid = lax.axis_index("c")
        sid = lax.axis_index("s")
        wid = sid * NC + cid

        pltpu.sync_copy(src_h.at[wid], gidx_v)
        pltpu.sync_copy(typ_h.at[wid], typ_v)

        def build(i, _):
            for j in range(C // 16):
                sl = pl.ds(j * 16, 16)
                gidx_v[i, sl] = typ_v[i, sl] * N + gidx_v[i, sl]
            return 0

        lax.fori_loop(0, NCH, build, 0)
        pltpu.sync_copy(gidx_v, gidx_h.at[wid])

    return k(src2d, typ2d)


def _sc_ea_sum(edge_attr, dst2d):
    mesh = plsc.VectorSubcoreMesh(core_axis_name="c", subcore_axis_name="s")

    @functools.partial(
        pl.kernel,
        out_type=[
            jax.ShapeDtypeStruct((NP, D), _f32),
            jax.ShapeDtypeStruct((NP, D), _f32),
        ],
        mesh=mesh,
        scratch_types=[
            pltpu.VMEM((NCH, C), _i32),
            pltpu.VMEM((C, DE), _f32),
            pltpu.VMEM((C, D), _f32),
            pltpu.VMEM_SHARED((NP, D), _f32),
        ],
    )
    def k(ea_h, dst_h, u0_h, u1_h,
          dst_v, ebuf_v, erows_v, eacc):
        cid = lax.axis_index("c")
        sid = lax.axis_index("s")
        wid = sid * NC + cid

        pltpu.sync_copy(dst_h.at[wid], dst_v)

        _zero_vmem_f32(erows_v, C, D)
        _zero_stripe(erows_v, eacc, sid * NPT)
        plsc.subcore_barrier()

        def step(i, _):
            ebase = (wid * NCH + i) * C
            pltpu.sync_copy(ea_h.at[pl.ds(ebase, C)], ebuf_v)

            def widen(r, _):
                erows_v[r, pl.ds(0, 16)] = ebuf_v[r, pl.ds(0, 16)]
                return 0

            lax.fori_loop(0, C, widen, 0)
            pltpu.sync_copy(erows_v, eacc.at[dst_v.at[i]], add=True)
            return 0

        lax.fori_loop(0, NCH, step, 0)
        plsc.subcore_barrier()

        base = sid * NPT

        @pl.when(cid == 0)
        def _():
            pltpu.sync_copy(eacc.at[pl.ds(base, NPT)], u0_h.at[pl.ds(base, NPT)])

        @pl.when(cid == 1)
        def _():
            pltpu.sync_copy(eacc.at[pl.ds(base, NPT)], u1_h.at[pl.ds(base, NPT)])

    return k(edge_attr, dst2d)


def _sc_layer(table, gidx2d, dst2d):
    mesh = plsc.VectorSubcoreMesh(core_axis_name="c", subcore_axis_name="s")

    @functools.partial(
        pl.kernel,
        out_type=[
            jax.ShapeDtypeStruct((NP, D), _f32),
            jax.ShapeDtypeStruct((NP, D), _f32),
        ],
        mesh=mesh,
        scratch_types=[
            pltpu.VMEM((NCH, C), _i32),
            pltpu.VMEM((NCH, C), _i32),
            pltpu.VMEM((C, D), _f32),
            pltpu.VMEM_SHARED((NP, D), _f32),
            pltpu.SemaphoreType.DMA,
        ],
    )
    def k(table_h, gidx_h, dst_h, agg0_h, agg1_h,
          gidx_v, dst_v, rows_v, acc, sem):
        cid = lax.axis_index("c")
        sid = lax.axis_index("s")
        wid = sid * NC + cid

        pltpu.sync_copy(gidx_h.at[wid], gidx_v)
        pltpu.sync_copy(dst_h.at[wid], dst_v)

        _zero_vmem_f32(rows_v, C, D)
        _zero_stripe(rows_v, acc, sid * NPT)
        plsc.subcore_barrier()

        def step(i, _):
            pltpu.async_copy(table_h.at[gidx_v.at[i]], rows_v, sem).wait()
            pltpu.sync_copy(rows_v, acc.at[dst_v.at[i]], add=True)
            return 0

        lax.fori_loop(0, NCH, step, 0)
        plsc.subcore_barrier()

        base = sid * NPT

        @pl.when(cid == 0)
        def _():
            pltpu.sync_copy(acc.at[pl.ds(base, NPT)], agg0_h.at[pl.ds(base, NPT)])

        @pl.when(cid == 1)
        def _():
            pltpu.sync_copy(acc.at[pl.ds(base, NPT)], agg1_h.at[pl.ds(base, NPT)])

    return k(table, gidx2d, dst2d)


def _sc_label_gather(x, h1, agg10, agg11, eaw1, labels):
    mesh = plsc.VectorSubcoreMesh(core_axis_name="c", subcore_axis_name="s")
    LPW = L // NW

    @functools.partial(
        pl.kernel,
        out_type=[
            jax.ShapeDtypeStruct((L, D), _f32),
            jax.ShapeDtypeStruct((L, D), _f32),
            jax.ShapeDtypeStruct((L, D), _f32),
            jax.ShapeDtypeStruct((L, D), _f32),
            jax.ShapeDtypeStruct((L, D), _f32),
        ],
        mesh=mesh,
        scratch_types=[
            pltpu.VMEM((LPW,), _i32),
            pltpu.VMEM((LPW, D), _f32),
            pltpu.SemaphoreType.DMA,
        ],
    )
    def k(x_h, h1_h, a0_h, a1_h, e_h, lab_h,
          xg_h, hg_h, ag0_h, ag1_h, eg_h,
          lab_v, buf_v, sem):
        cid = lax.axis_index("c")
        sid = lax.axis_index("s")
        wid = sid * NC + cid
        base = wid * LPW
        pltpu.sync_copy(lab_h.at[pl.ds(base, LPW)], lab_v)
        for src_h, out_h in ((x_h, xg_h), (h1_h, hg_h), (a0_h, ag0_h),
                             (a1_h, ag1_h), (e_h, eg_h)):
            pltpu.async_copy(src_h.at[lab_v], buf_v, sem).wait()
            pltpu.sync_copy(buf_v, out_h.at[pl.ds(base, LPW)])

    return k(x, h1, agg10, agg11, eaw1, labels)


def _tc_rel_table(x, W_rel):
    BN = 1000

    def body(x_ref, w_ref, out_ref):
        out_ref[0] = jnp.dot(x_ref[...], w_ref[0], preferred_element_type=_f32)

    return pl.pallas_call(
        body,
        grid=(N // BN, R),
        in_specs=[
            pl.BlockSpec((BN, D), lambda nb, r: (nb, 0)),
            pl.BlockSpec((1, D, D), lambda nb, r: (r, 0, 0)),
        ],
        out_specs=pl.BlockSpec((1, BN, D), lambda nb, r: (r, nb, 0)),
        out_shape=jax.ShapeDtypeStruct((R, N, D), _f32),
    )(x, W_rel)


def _tc_layer_update(x, agg0, agg1, u0, u1, W_self, W_edge, b, W_rel_next,
                     W_edge_next):
    BN = 1000

    def body(x_ref, a0_ref, a1_ref, u0_ref, u1_ref, ws_ref, we_ref, b_ref,
             wr_ref, wen_ref, h_ref, t_ref, ew_ref):
        ea = u0_ref[:, :DE] + u1_ref[:, :DE]
        h = jnp.dot(x_ref[...], ws_ref[...], preferred_element_type=_f32)
        h = h + a0_ref[...] + a1_ref[...] + b_ref[...]
        h = h + jnp.dot(ea, we_ref[...], preferred_element_type=_f32)
        h = jnp.maximum(h, 0.0)
        h_ref[...] = h
        ew_ref[...] = jnp.dot(ea, wen_ref[...], preferred_element_type=_f32)
        for r in range(R):
            t_ref[r] = jnp.dot(h, wr_ref[r], preferred_element_type=_f32)

    return pl.pallas_call(
        body,
        grid=(N // BN,),
        in_specs=[
            pl.BlockSpec((BN, D), lambda nb: (nb, 0)),
            pl.BlockSpec((BN, D), lambda nb: (nb, 0)),
            pl.BlockSpec((BN, D), lambda nb: (nb, 0)),
            pl.BlockSpec((BN, D), lambda nb: (nb, 0)),
            pl.BlockSpec((BN, D), lambda nb: (nb, 0)),
            pl.BlockSpec((D, D), lambda nb: (0, 0)),
            pl.BlockSpec((DE, D), lambda nb: (0, 0)),
            pl.BlockSpec((1, D), lambda nb: (0, 0)),
            pl.BlockSpec((R, D, D), lambda nb: (0, 0, 0)),
            pl.BlockSpec((DE, D), lambda nb: (0, 0)),
        ],
        out_specs=[
            pl.BlockSpec((BN, D), lambda nb: (nb, 0)),
            pl.BlockSpec((R, BN, D), lambda nb: (0, nb, 0)),
            pl.BlockSpec((BN, D), lambda nb: (nb, 0)),
        ],
        out_shape=[
            jax.ShapeDtypeStruct((N, D), _f32),
            jax.ShapeDtypeStruct((R, N, D), _f32),
            jax.ShapeDtypeStruct((N, D), _f32),
        ],
    )(x, agg0, agg1, u0, u1, W_self, W_edge, b, W_rel_next, W_edge_next)


def _tc_head(xg, hg, ag0, ag1, eg, W_self1, b1,
             Wq, Wk, Wv, w1, bm1, w2, bm2, w3, bm3):
    def body(xg_ref, hg_ref, a0_ref, a1_ref, eg_ref, ws_ref,
             b_ref, wq_ref, wk_ref, wv_ref, w1_ref, b1_ref, w2_ref, b2_ref,
             w3_ref, b3_ref, out_ref):
        h2 = jnp.dot(hg_ref[...], ws_ref[...], preferred_element_type=_f32)
        h2 = h2 + a0_ref[...] + a1_ref[...] + eg_ref[...] + b_ref[...]
        h2 = jnp.maximum(h2, 0.0)

        qh = jnp.dot(xg_ref[...], wq_ref[...], preferred_element_type=_f32)
        kh = jnp.dot(h2, wk_ref[...], preferred_element_type=_f32)
        vh = jnp.dot(h2, wv_ref[...], preferred_element_type=_f32)
        score = jnp.sum(qh * kh, axis=-1, keepdims=True) / jnp.sqrt(_f32(D))
        attn = jax.nn.sigmoid(score) * vh

        z = jnp.dot(attn, w1_ref[...], preferred_element_type=_f32) + b1_ref[...]
        z = jnp.maximum(z, 0.0)
        z = jnp.dot(z, w2_ref[...], preferred_element_type=_f32) + b2_ref[...]
        z = jnp.maximum(z, 0.0)
        logit = jnp.dot(z, w3_ref[...], preferred_element_type=_f32) + b3_ref[...]
        out_ref[...] = jnp.broadcast_to(logit, (L, D))

    return pl.pallas_call(
        body,
        out_shape=jax.ShapeDtypeStruct((L, D), _f32),
    )(xg, hg, ag0, ag1, eg, W_self1, b1,
      Wq, Wk, Wv, w1, bm1, w2, bm2, w3, bm3)


def kernel(x, edge_index, edge_attr, edge_type, label_indices,
           correct_label_mask, W_rel0, W_self0, W_edge0, b0, W_rel1, W_self1,
           W_edge1, b1, Wq, Wk, Wv, mlp_w1, mlp_b1, mlp_w2, mlp_b2, mlp_w3,
           mlp_b3):
    src2d = edge_index[0].reshape(NW, NCH, C)
    dst2d = edge_index[1].reshape(NW, NCH, C)
    typ2d = edge_type.reshape(NW, NCH, C)

    gidx2d = _sc_gidx(src2d, typ2d)
    u0, u1 = _sc_ea_sum(edge_attr, dst2d)
    table0 = _tc_rel_table(x, W_rel0).reshape(R * N, D)
    agg00, agg01 = _sc_layer(table0, gidx2d, dst2d)

    h1, table1, eaw1 = _tc_layer_update(
        x, agg00, agg01, u0, u1, W_self0, W_edge0, b0.reshape(1, D), W_rel1,
        W_edge1)
    agg10, agg11 = _sc_layer(table1.reshape(R * N, D), gidx2d, dst2d)

    xg, hg, ag0, ag1, eg = _sc_label_gather(
        x, h1, agg10, agg11, eaw1, label_indices)

    out = _tc_head(
        xg, hg, ag0, ag1, eg, W_self1, b1.reshape(1, D),
        Wq, Wk, Wv, mlp_w1, mlp_b1.reshape(1, 2 * D), mlp_w2,
        mlp_b2.reshape(1, D), mlp_w3, mlp_b3.reshape(1, 1))
    logits = out[:, :1]
    return (logits, correct_label_mask)
